# baseline (speedup 1.0000x reference)
import functools

import jax
import jax.numpy as jnp
from jax import lax
from jax.experimental import pallas as pl
from jax.experimental.pallas import tpu as pltpu

N_ROWS_LIN = 400
N_ROWS_DIST = 200
LANES = 128


def _ceil_to(x, m):
    return (x + m - 1) // m * m


def _linear_kernel(x_ref, wt_ref, b_ref, o_ref, *, act):
    y = jnp.dot(x_ref[...], wt_ref[...], preferred_element_type=jnp.float32)
    y = y + b_ref[...]
    if act == "relu":
        y = jnp.maximum(y, 0.0)
    o_ref[...] = y


def _linear(x, w, b, act=None, row_tile=N_ROWS_LIN):
    n, cin = x.shape
    cout = w.shape[0]
    assert n % row_tile == 0, (n, row_tile)
    wt = w.T
    b2 = b.reshape(1, cout)
    return pl.pallas_call(
        functools.partial(_linear_kernel, act=act),
        grid=(n // row_tile,),
        in_specs=[
            pl.BlockSpec((row_tile, cin), lambda i: (i, 0)),
            pl.BlockSpec((cin, cout), lambda i: (0, 0)),
            pl.BlockSpec((1, cout), lambda i: (0, 0)),
        ],
        out_specs=pl.BlockSpec((row_tile, cout), lambda i: (i, 0)),
        out_shape=jax.ShapeDtypeStruct((n, cout), jnp.float32),
    )(x, wt, b2)


def _dist_kernel(x_ref, xt_ref, o_ref, *, n_valid):
    x = x_ref[...]
    xt = xt_ref[...]
    dot = jnp.dot(x, xt, preferred_element_type=jnp.float32)
    sq_i = jnp.sum(x * x, axis=1, keepdims=True)
    sq_j = jnp.sum(xt * xt, axis=0, keepdims=True)
    neg = -((sq_i + sq_j) - 2.0 * dot)
    col = lax.broadcasted_iota(jnp.int32, neg.shape, 1)
    o_ref[...] = jnp.where(col < n_valid, neg, -1e30)


def _neg_dists(x):
    n, c = x.shape
    np_ = _ceil_to(n, LANES)
    xt = jnp.zeros((c, np_), jnp.float32).at[:, :n].set(x.T)
    return pl.pallas_call(
        functools.partial(_dist_kernel, n_valid=n),
        grid=(n // N_ROWS_DIST,),
        in_specs=[
            pl.BlockSpec((N_ROWS_DIST, c), lambda i: (i, 0)),
            pl.BlockSpec((c, np_), lambda i: (0, 0)),
        ],
        out_specs=pl.BlockSpec((N_ROWS_DIST, np_), lambda i: (i, 0)),
        out_shape=jax.ShapeDtypeStruct((n, np_), jnp.float32),
    )(x, xt)


def _knn(x, k):
    neg = _neg_dists(x)
    _, idx = lax.top_k(neg, k)
    return idx


def _edgeconv(p, idx, x):
    n, cin = x.shape
    k = idx.shape[1]
    cout = p["theta"]["W"].shape[0]
    diff = x[idx] - x[:, None, :]
    msg = _linear(diff.reshape(n * k, cin), p["theta"]["W"], p["theta"]["b"])
    phi = _linear(x, p["phi"]["W"], p["phi"]["b"])
    m = msg.reshape(n, k, cout) + phi[:, None, :]
    return jnp.max(m, axis=1)


def _gatv2(p, idx, h, heads=4, out=256):
    n = h.shape[0]
    fs = _linear(h, p["src"]["W"], p["src"]["b"]).reshape(n, heads, out)
    fd = _linear(h, p["dst"]["W"], p["dst"]["b"]).reshape(n, heads, out)
    fsn = fs[idx]
    e = jax.nn.leaky_relu(fsn + fd[:, None], negative_slope=0.2)
    logits = jnp.sum(e * p["attn"][None, None], axis=-1)
    a = jax.nn.softmax(logits, axis=1)
    o = jnp.sum(a[..., None] * fsn, axis=1).reshape(n, heads * out)
    return o + p["bias"]


def _bn(p, x, eps=1e-5):
    m = jnp.mean(x, axis=0)
    v = jnp.var(x, axis=0)
    return p["gamma"] * (x - m) / jnp.sqrt(v + eps) + p["beta"]


def _tnet(p, x):
    h = _linear(x, p["fc1"]["W"], p["fc1"]["b"], act="relu")
    h = _linear(h, p["fc2"]["W"], p["fc2"]["b"])
    h = jnp.max(h, axis=0, keepdims=True)
    h = jax.nn.relu(h @ p["fc3"]["W"].T + p["fc3"]["b"])
    h = (h @ p["fc4"]["W"].T + p["fc4"]["b"]).reshape(3, 3)
    q, r = jnp.linalg.qr(h)
    return q, r


def kernel(pointcloud, params):
    t, r = _tnet(params["tnet"], pointcloud)
    x = pointcloud @ t
    idx3d = _knn(x, 30)
    h0 = _edgeconv(params["G1"], idx3d, x)
    h0 = _edgeconv(params["G2"], idx3d, h0)
    g0 = _knn(h0, 30)
    h1 = _edgeconv(params["G3"], g0, h0)
    h1 = _edgeconv(params["G4"], g0, h1)
    g1 = _knn(h1, 30)
    h2 = _edgeconv(params["G5"], g1, h1)
    h = jnp.concatenate([h0, h1, h2], axis=1)
    h_ = _linear(h, params["fc0"]["W"], params["fc0"]["b"])
    h_ = jnp.max(h_, axis=0, keepdims=True)
    h_ = jnp.tile(h_, (x.shape[0], 1))
    hxy = _edgeconv(params["E1xy"], _knn(x[:, 0:2], 30), x)
    hxy_ = _edgeconv(params["E2xy"], _knn(hxy, 100), hxy)
    hyz = _edgeconv(params["E1yz"], _knn(x[:, 1:3], 30), x)
    hyz_ = _edgeconv(params["E2yz"], _knn(hyz, 100), hyz)
    hxz = _edgeconv(params["E1xz"], _knn(x[:, ::2], 30), x)
    hxz_ = _edgeconv(params["E2xz"], _knn(hxz, 100), hxz)
    h = jnp.concatenate([h, h_, hxy_, hyz_, hxz_], axis=1)
    h = _gatv2(params["gat"], idx3d, h)
    h = jax.nn.relu(_bn(params["bn1"], _linear(h, params["fc1"]["W"], params["fc1"]["b"])))
    h = jax.nn.relu(_bn(params["bn2"], _linear(h, params["fc2"]["W"], params["fc2"]["b"])))
    h = _linear(h, params["fc3"]["W"], params["fc3"]["b"])
    return (h, r)


# speedup vs baseline: 3.4955x; 3.4955x over previous
import functools

import jax
import jax.numpy as jnp
from jax import lax
from jax.experimental import pallas as pl
from jax.experimental.pallas import tpu as pltpu

N_ROWS_LIN = 400
N_ROWS_DIST = 200
LANES = 128


def _ceil_to(x, m):
    return (x + m - 1) // m * m


def _linear_kernel(x_ref, wt_ref, b_ref, o_ref, *, act):
    y = jnp.dot(x_ref[...], wt_ref[...], preferred_element_type=jnp.float32)
    y = y + b_ref[...]
    if act == "relu":
        y = jnp.maximum(y, 0.0)
    o_ref[...] = y


def _linear(x, w, b, act=None, row_tile=N_ROWS_LIN):
    n, cin = x.shape
    cout = w.shape[0]
    assert n % row_tile == 0, (n, row_tile)
    wt = w.T
    b2 = b.reshape(1, cout)
    return pl.pallas_call(
        functools.partial(_linear_kernel, act=act),
        grid=(n // row_tile,),
        in_specs=[
            pl.BlockSpec((row_tile, cin), lambda i: (i, 0)),
            pl.BlockSpec((cin, cout), lambda i: (0, 0)),
            pl.BlockSpec((1, cout), lambda i: (0, 0)),
        ],
        out_specs=pl.BlockSpec((row_tile, cout), lambda i: (i, 0)),
        out_shape=jax.ShapeDtypeStruct((n, cout), jnp.float32),
    )(x, wt, b2)


def _dist_kernel(x_ref, xt_ref, o_ref, *, n_valid):
    x = x_ref[...]
    xt = xt_ref[...]
    dot = jnp.dot(x, xt, preferred_element_type=jnp.float32)
    sq_i = jnp.sum(x * x, axis=1, keepdims=True)
    sq_j = jnp.sum(xt * xt, axis=0, keepdims=True)
    neg = -((sq_i + sq_j) - 2.0 * dot)
    col = lax.broadcasted_iota(jnp.int32, neg.shape, 1)
    o_ref[...] = jnp.where(col < n_valid, neg, -1e30)


def _neg_dists(x):
    n, c = x.shape
    np_ = _ceil_to(n, LANES)
    xt = jnp.zeros((c, np_), jnp.float32).at[:, :n].set(x.T)
    return pl.pallas_call(
        functools.partial(_dist_kernel, n_valid=n),
        grid=(n // N_ROWS_DIST,),
        in_specs=[
            pl.BlockSpec((N_ROWS_DIST, c), lambda i: (i, 0)),
            pl.BlockSpec((c, np_), lambda i: (0, 0)),
        ],
        out_specs=pl.BlockSpec((N_ROWS_DIST, np_), lambda i: (i, 0)),
        out_shape=jax.ShapeDtypeStruct((n, np_), jnp.float32),
    )(x, xt)


def _knn(x, k):
    return _knn_fused(x, k)


_I32_MIN = -2147483648


def _sortable_keys(neg):
    b = lax.bitcast_convert_type(neg, jnp.int32)
    return jnp.where(b < 0, b ^ 0x7FFFFFFF, b)


def _knn_sel_kernel(x_ref, xt_ref, o_ref, *, n_valid, k, m, r_blk):
    x = x_ref[...]
    xt = xt_ref[...]
    npad = xt.shape[1]
    nc = npad // LANES
    dot = jnp.dot(x, xt, preferred_element_type=jnp.float32)
    sq_i = jnp.sum(x * x, axis=1, keepdims=True)
    sq_j = jnp.sum(xt * xt, axis=0, keepdims=True)
    neg = -((sq_i + sq_j) - 2.0 * dot)
    col = lax.broadcasted_iota(jnp.int32, neg.shape, 1)
    neg = jnp.where(col < n_valid, neg, -jnp.inf)
    key = _sortable_keys(neg)

    key3 = key.reshape(r_blk, nc, LANES)
    lane = lax.broadcasted_iota(jnp.int32, (r_blk, nc, LANES), 2)
    cbase = lax.broadcasted_iota(jnp.int32, (r_blk, nc), 1) * LANES
    cur = key3
    cvals, cidxs = [], []
    for _ in range(m):
        cmax = jnp.max(cur, axis=-1)
        amax = jnp.min(jnp.where(cur == cmax[..., None], lane, LANES), axis=-1)
        cvals.append(cmax)
        cidxs.append(cbase + amax)
        cur = jnp.where(lane == amax[..., None], _I32_MIN, cur)
    cand = jnp.concatenate(cvals, axis=1)
    cidx = jnp.concatenate(cidxs, axis=1)

    lo = jnp.full((r_blk, 1), _I32_MIN, jnp.int32)
    hi = jnp.full((r_blk, 1), 2147483647, jnp.int32)

    def bs_body(_, lh):
        lo, hi = lh
        d = (hi.astype(jnp.uint32) - lo.astype(jnp.uint32)) >> 1
        mid = (lo.astype(jnp.uint32) + d).astype(jnp.int32)
        cnt = jnp.sum((cand >= mid).astype(jnp.int32), axis=1, keepdims=True)
        ge = cnt >= k
        return jnp.where(ge, mid, lo), jnp.where(ge, hi, mid - 1)

    lo, hi = lax.fori_loop(0, 32, bs_body, (lo, hi))
    tau = lo

    cnt_full = jnp.sum((key >= tau).astype(jnp.int32), axis=1, keepdims=True)
    all_ok = jnp.all(cnt_full == k)

    p_iota = lax.broadcasted_iota(jnp.int32, (r_blk, k), 1)

    def fast_path():
        mask = cand >= tau
        w = cand.shape[1]
        tri = (lax.broadcasted_iota(jnp.int32, (w, w), 0)
               <= lax.broadcasted_iota(jnp.int32, (w, w), 1)).astype(jnp.float32)
        ranks = jnp.dot(mask.astype(jnp.float32), tri,
                        preferred_element_type=jnp.float32)
        ranks = jnp.where(mask, ranks, 0.0).astype(jnp.int32)
        cols = [jnp.sum(jnp.where(ranks == p + 1, cidx, 0), axis=1,
                        keepdims=True) for p in range(k)]
        return jnp.concatenate(cols, axis=1)

    def slow_path():
        colf = lax.broadcasted_iota(jnp.int32, key.shape, 1)

        def body(p, carry):
            kcur, out = carry
            vmax = jnp.max(kcur, axis=1, keepdims=True)
            j = jnp.min(jnp.where(kcur == vmax, colf, npad), axis=1,
                        keepdims=True)
            out = jnp.where(p_iota == p, j, out)
            kcur = jnp.where(colf == j, _I32_MIN, kcur)
            return kcur, out

        _, out = lax.fori_loop(0, k, body, (key, jnp.zeros((r_blk, k), jnp.int32)))
        return out

    o_ref[...] = lax.cond(all_ok, fast_path, slow_path)


def _knn_fused(x, k):
    n, c = x.shape
    np_ = _ceil_to(n, LANES)
    m = 6 if k <= 30 else 12
    r_blk = 200
    xt = jnp.zeros((c, np_), jnp.float32).at[:, :n].set(x.T)
    return pl.pallas_call(
        functools.partial(_knn_sel_kernel, n_valid=n, k=k, m=m, r_blk=r_blk),
        grid=(n // r_blk,),
        in_specs=[
            pl.BlockSpec((r_blk, c), lambda i: (i, 0)),
            pl.BlockSpec((c, np_), lambda i: (0, 0)),
        ],
        out_specs=pl.BlockSpec((r_blk, k), lambda i: (i, 0)),
        out_shape=jax.ShapeDtypeStruct((n, k), jnp.int32),
    )(x, xt)


def _edgeconv(p, idx, x):
    n, cin = x.shape
    k = idx.shape[1]
    cout = p["theta"]["W"].shape[0]
    diff = x[idx] - x[:, None, :]
    msg = _linear(diff.reshape(n * k, cin), p["theta"]["W"], p["theta"]["b"])
    phi = _linear(x, p["phi"]["W"], p["phi"]["b"])
    m = msg.reshape(n, k, cout) + phi[:, None, :]
    return jnp.max(m, axis=1)


def _gatv2(p, idx, h, heads=4, out=256):
    n = h.shape[0]
    fs = _linear(h, p["src"]["W"], p["src"]["b"]).reshape(n, heads, out)
    fd = _linear(h, p["dst"]["W"], p["dst"]["b"]).reshape(n, heads, out)
    fsn = fs[idx]
    e = jax.nn.leaky_relu(fsn + fd[:, None], negative_slope=0.2)
    logits = jnp.sum(e * p["attn"][None, None], axis=-1)
    a = jax.nn.softmax(logits, axis=1)
    o = jnp.sum(a[..., None] * fsn, axis=1).reshape(n, heads * out)
    return o + p["bias"]


def _bn(p, x, eps=1e-5):
    m = jnp.mean(x, axis=0)
    v = jnp.var(x, axis=0)
    return p["gamma"] * (x - m) / jnp.sqrt(v + eps) + p["beta"]


def _tnet(p, x):
    h = _linear(x, p["fc1"]["W"], p["fc1"]["b"], act="relu")
    h = _linear(h, p["fc2"]["W"], p["fc2"]["b"])
    h = jnp.max(h, axis=0, keepdims=True)
    h = jax.nn.relu(h @ p["fc3"]["W"].T + p["fc3"]["b"])
    h = (h @ p["fc4"]["W"].T + p["fc4"]["b"]).reshape(3, 3)
    q, r = jnp.linalg.qr(h)
    return q, r


def kernel(pointcloud, params):
    t, r = _tnet(params["tnet"], pointcloud)
    x = pointcloud @ t
    idx3d = _knn(x, 30)
    h0 = _edgeconv(params["G1"], idx3d, x)
    h0 = _edgeconv(params["G2"], idx3d, h0)
    g0 = _knn(h0, 30)
    h1 = _edgeconv(params["G3"], g0, h0)
    h1 = _edgeconv(params["G4"], g0, h1)
    g1 = _knn(h1, 30)
    h2 = _edgeconv(params["G5"], g1, h1)
    h = jnp.concatenate([h0, h1, h2], axis=1)
    h_ = _linear(h, params["fc0"]["W"], params["fc0"]["b"])
    h_ = jnp.max(h_, axis=0, keepdims=True)
    h_ = jnp.tile(h_, (x.shape[0], 1))
    hxy = _edgeconv(params["E1xy"], _knn(x[:, 0:2], 30), x)
    hxy_ = _edgeconv(params["E2xy"], _knn(hxy, 100), hxy)
    hyz = _edgeconv(params["E1yz"], _knn(x[:, 1:3], 30), x)
    hyz_ = _edgeconv(params["E2yz"], _knn(hyz, 100), hyz)
    hxz = _edgeconv(params["E1xz"], _knn(x[:, ::2], 30), x)
    hxz_ = _edgeconv(params["E2xz"], _knn(hxz, 100), hxz)
    h = jnp.concatenate([h, h_, hxy_, hyz_, hxz_], axis=1)
    h = _gatv2(params["gat"], idx3d, h)
    h = jax.nn.relu(_bn(params["bn1"], _linear(h, params["fc1"]["W"], params["fc1"]["b"])))
    h = jax.nn.relu(_bn(params["bn2"], _linear(h, params["fc2"]["W"], params["fc2"]["b"])))
    h = _linear(h, params["fc3"]["W"], params["fc3"]["b"])
    return (h, r)


# speedup vs baseline: 3.6997x; 1.0584x over previous
import functools

import jax
import jax.numpy as jnp
from jax import lax
from jax.experimental import pallas as pl
from jax.experimental.pallas import tpu as pltpu
from jax.experimental.pallas import tpu_sc as plsc

N_ROWS_LIN = 400
N_ROWS_DIST = 200
LANES = 128


def _ceil_to(x, m):
    return (x + m - 1) // m * m


def _linear_kernel(x_ref, wt_ref, b_ref, o_ref, *, act):
    y = jnp.dot(x_ref[...], wt_ref[...], preferred_element_type=jnp.float32)
    y = y + b_ref[...]
    if act == "relu":
        y = jnp.maximum(y, 0.0)
    o_ref[...] = y


def _linear(x, w, b, act=None, row_tile=N_ROWS_LIN):
    n, cin = x.shape
    cout = w.shape[0]
    assert n % row_tile == 0, (n, row_tile)
    wt = w.T
    b2 = b.reshape(1, cout)
    return pl.pallas_call(
        functools.partial(_linear_kernel, act=act),
        grid=(n // row_tile,),
        in_specs=[
            pl.BlockSpec((row_tile, cin), lambda i: (i, 0)),
            pl.BlockSpec((cin, cout), lambda i: (0, 0)),
            pl.BlockSpec((1, cout), lambda i: (0, 0)),
        ],
        out_specs=pl.BlockSpec((row_tile, cout), lambda i: (i, 0)),
        out_shape=jax.ShapeDtypeStruct((n, cout), jnp.float32),
    )(x, wt, b2)


def _dist_kernel(x_ref, xt_ref, o_ref, *, n_valid):
    x = x_ref[...]
    xt = xt_ref[...]
    dot = jnp.dot(x, xt, preferred_element_type=jnp.float32)
    sq_i = jnp.sum(x * x, axis=1, keepdims=True)
    sq_j = jnp.sum(xt * xt, axis=0, keepdims=True)
    neg = -((sq_i + sq_j) - 2.0 * dot)
    col = lax.broadcasted_iota(jnp.int32, neg.shape, 1)
    o_ref[...] = jnp.where(col < n_valid, neg, -1e30)


def _neg_dists(x):
    n, c = x.shape
    np_ = _ceil_to(n, LANES)
    xt = jnp.zeros((c, np_), jnp.float32).at[:, :n].set(x.T)
    return pl.pallas_call(
        functools.partial(_dist_kernel, n_valid=n),
        grid=(n // N_ROWS_DIST,),
        in_specs=[
            pl.BlockSpec((N_ROWS_DIST, c), lambda i: (i, 0)),
            pl.BlockSpec((c, np_), lambda i: (0, 0)),
        ],
        out_specs=pl.BlockSpec((N_ROWS_DIST, np_), lambda i: (i, 0)),
        out_shape=jax.ShapeDtypeStruct((n, np_), jnp.float32),
    )(x, xt)


def _knn(x, k):
    return _knn_fused(x, k)


_I32_MIN = -2147483648


def _sortable_keys(neg):
    b = lax.bitcast_convert_type(neg, jnp.int32)
    return jnp.where(b < 0, b ^ 0x7FFFFFFF, b)


def _knn_sel_kernel(x_ref, xt_ref, o_ref, *, n_valid, k, m, r_blk):
    x = x_ref[...]
    xt = xt_ref[...]
    npad = xt.shape[1]
    nc = npad // LANES
    dot = jnp.dot(x, xt, preferred_element_type=jnp.float32)
    sq_i = jnp.sum(x * x, axis=1, keepdims=True)
    sq_j = jnp.sum(xt * xt, axis=0, keepdims=True)
    neg = -((sq_i + sq_j) - 2.0 * dot)
    col = lax.broadcasted_iota(jnp.int32, neg.shape, 1)
    neg = jnp.where(col < n_valid, neg, -jnp.inf)
    key = _sortable_keys(neg)

    key3 = key.reshape(r_blk, nc, LANES)
    lane = lax.broadcasted_iota(jnp.int32, (r_blk, nc, LANES), 2)
    cbase = lax.broadcasted_iota(jnp.int32, (r_blk, nc), 1) * LANES
    cur = key3
    cvals, cidxs = [], []
    for _ in range(m):
        cmax = jnp.max(cur, axis=-1)
        amax = jnp.min(jnp.where(cur == cmax[..., None], lane, LANES), axis=-1)
        cvals.append(cmax)
        cidxs.append(cbase + amax)
        cur = jnp.where(lane == amax[..., None], _I32_MIN, cur)
    cand = jnp.concatenate(cvals, axis=1)
    cidx = jnp.concatenate(cidxs, axis=1)

    lo = jnp.full((r_blk, 1), _I32_MIN, jnp.int32)
    hi = jnp.full((r_blk, 1), 2147483647, jnp.int32)

    def bs_body(_, lh):
        lo, hi = lh
        d = (hi.astype(jnp.uint32) - lo.astype(jnp.uint32)) >> 1
        mid = (lo.astype(jnp.uint32) + d).astype(jnp.int32)
        cnt = jnp.sum((cand >= mid).astype(jnp.int32), axis=1, keepdims=True)
        ge = cnt >= k
        return jnp.where(ge, mid, lo), jnp.where(ge, hi, mid - 1)

    lo, hi = lax.fori_loop(0, 32, bs_body, (lo, hi))
    tau = lo

    cnt_full = jnp.sum((key >= tau).astype(jnp.int32), axis=1, keepdims=True)
    all_ok = jnp.all(cnt_full == k)

    p_iota = lax.broadcasted_iota(jnp.int32, (r_blk, k), 1)

    def fast_path():
        mask = cand >= tau
        w = cand.shape[1]
        tri = (lax.broadcasted_iota(jnp.int32, (w, w), 0)
               <= lax.broadcasted_iota(jnp.int32, (w, w), 1)).astype(jnp.float32)
        ranks = jnp.dot(mask.astype(jnp.float32), tri,
                        preferred_element_type=jnp.float32)
        ranks = jnp.where(mask, ranks, 0.0).astype(jnp.int32)
        cols = [jnp.sum(jnp.where(ranks == p + 1, cidx, 0), axis=1,
                        keepdims=True) for p in range(k)]
        return jnp.concatenate(cols, axis=1)

    def slow_path():
        colf = lax.broadcasted_iota(jnp.int32, key.shape, 1)

        def body(p, carry):
            kcur, out = carry
            vmax = jnp.max(kcur, axis=1, keepdims=True)
            j = jnp.min(jnp.where(kcur == vmax, colf, npad), axis=1,
                        keepdims=True)
            out = jnp.where(p_iota == p, j, out)
            kcur = jnp.where(colf == j, _I32_MIN, kcur)
            return kcur, out

        _, out = lax.fori_loop(0, k, body, (key, jnp.zeros((r_blk, k), jnp.int32)))
        return out

    o_ref[...] = lax.cond(all_ok, fast_path, slow_path)


def _knn_fused(x, k):
    n, c = x.shape
    np_ = _ceil_to(n, LANES)
    m = 6 if k <= 30 else 12
    r_blk = 200
    xt = jnp.zeros((c, np_), jnp.float32).at[:, :n].set(x.T)
    return pl.pallas_call(
        functools.partial(_knn_sel_kernel, n_valid=n, k=k, m=m, r_blk=r_blk),
        grid=(n // r_blk,),
        in_specs=[
            pl.BlockSpec((r_blk, c), lambda i: (i, 0)),
            pl.BlockSpec((c, np_), lambda i: (0, 0)),
        ],
        out_specs=pl.BlockSpec((r_blk, k), lambda i: (i, 0)),
        out_shape=jax.ShapeDtypeStruct((n, k), jnp.int32),
    )(x, xt)


_SC_WORKERS = 32


def _sc_gather(table, idx):
    v, d0 = table.shape
    d = _ceil_to(d0, LANES)
    if d != d0:
        table = jnp.zeros((v, d), table.dtype).at[:, :d0].set(table)
    b = idx.shape[0]
    chunk = max(64, min(1024, (100000 // d) // 8 * 8))
    blk = _SC_WORKERS * chunk
    b_pad = _ceil_to(b, blk)
    bpw = b_pad // _SC_WORKERS
    nch = bpw // chunk
    idx_pad = jnp.zeros((b_pad,), jnp.int32).at[:b].set(idx)

    mesh = plsc.VectorSubcoreMesh(core_axis_name="c", subcore_axis_name="s")

    @functools.partial(
        pl.kernel, mesh=mesh,
        out_type=jax.ShapeDtypeStruct((b_pad, d), jnp.float32),
        scratch_types=[
            pltpu.VMEM((chunk,), jnp.int32),
            pltpu.VMEM((chunk, d), jnp.float32),
            pltpu.SemaphoreType.DMA,
        ],
    )
    def gk(table_hbm, idx_hbm, out_hbm, idx_v, rows_v, sem):
        wid = lax.axis_index("s") * 2 + lax.axis_index("c")
        base = wid * bpw

        def body(ci, carry):
            off = base + ci * chunk
            pltpu.sync_copy(idx_hbm.at[pl.ds(off, chunk)], idx_v)
            pltpu.async_copy(table_hbm.at[idx_v], rows_v, sem).wait()
            pltpu.sync_copy(rows_v, out_hbm.at[pl.ds(off, chunk)])
            return carry

        lax.fori_loop(0, nch, body, 0)

    return gk(table, idx_pad)[:b, :d0]


def _neighbor_rows(x, idx):
    n, k = idx.shape
    if x.shape[1] >= 64:
        return _sc_gather(x, idx.reshape(-1)).reshape(n, k, x.shape[1])
    return x[idx]


def _edgeconv(p, idx, x):
    n, cin = x.shape
    k = idx.shape[1]
    cout = p["theta"]["W"].shape[0]
    diff = _neighbor_rows(x, idx) - x[:, None, :]
    msg = _linear(diff.reshape(n * k, cin), p["theta"]["W"], p["theta"]["b"])
    phi = _linear(x, p["phi"]["W"], p["phi"]["b"])
    m = msg.reshape(n, k, cout) + phi[:, None, :]
    return jnp.max(m, axis=1)


def _gatv2(p, idx, h, heads=4, out=256):
    n = h.shape[0]
    k = idx.shape[1]
    fs = _linear(h, p["src"]["W"], p["src"]["b"])
    fd = _linear(h, p["dst"]["W"], p["dst"]["b"]).reshape(n, heads, out)
    fsn = _sc_gather(fs, idx.reshape(-1)).reshape(n, k, heads, out)
    e = jax.nn.leaky_relu(fsn + fd[:, None], negative_slope=0.2)
    logits = jnp.sum(e * p["attn"][None, None], axis=-1)
    a = jax.nn.softmax(logits, axis=1)
    o = jnp.sum(a[..., None] * fsn, axis=1).reshape(n, heads * out)
    return o + p["bias"]


def _bn(p, x, eps=1e-5):
    m = jnp.mean(x, axis=0)
    v = jnp.var(x, axis=0)
    return p["gamma"] * (x - m) / jnp.sqrt(v + eps) + p["beta"]


def _tnet(p, x):
    h = _linear(x, p["fc1"]["W"], p["fc1"]["b"], act="relu")
    h = _linear(h, p["fc2"]["W"], p["fc2"]["b"])
    h = jnp.max(h, axis=0, keepdims=True)
    h = jax.nn.relu(h @ p["fc3"]["W"].T + p["fc3"]["b"])
    h = (h @ p["fc4"]["W"].T + p["fc4"]["b"]).reshape(3, 3)
    q, r = jnp.linalg.qr(h)
    return q, r


def kernel(pointcloud, params):
    t, r = _tnet(params["tnet"], pointcloud)
    x = pointcloud @ t
    idx3d = _knn(x, 30)
    h0 = _edgeconv(params["G1"], idx3d, x)
    h0 = _edgeconv(params["G2"], idx3d, h0)
    g0 = _knn(h0, 30)
    h1 = _edgeconv(params["G3"], g0, h0)
    h1 = _edgeconv(params["G4"], g0, h1)
    g1 = _knn(h1, 30)
    h2 = _edgeconv(params["G5"], g1, h1)
    h = jnp.concatenate([h0, h1, h2], axis=1)
    h_ = _linear(h, params["fc0"]["W"], params["fc0"]["b"])
    h_ = jnp.max(h_, axis=0, keepdims=True)
    h_ = jnp.tile(h_, (x.shape[0], 1))
    hxy = _edgeconv(params["E1xy"], _knn(x[:, 0:2], 30), x)
    hxy_ = _edgeconv(params["E2xy"], _knn(hxy, 100), hxy)
    hyz = _edgeconv(params["E1yz"], _knn(x[:, 1:3], 30), x)
    hyz_ = _edgeconv(params["E2yz"], _knn(hyz, 100), hyz)
    hxz = _edgeconv(params["E1xz"], _knn(x[:, ::2], 30), x)
    hxz_ = _edgeconv(params["E2xz"], _knn(hxz, 100), hxz)
    h = jnp.concatenate([h, h_, hxy_, hyz_, hxz_], axis=1)
    h = _gatv2(params["gat"], idx3d, h)
    h = jax.nn.relu(_bn(params["bn1"], _linear(h, params["fc1"]["W"], params["fc1"]["b"])))
    h = jax.nn.relu(_bn(params["bn2"], _linear(h, params["fc2"]["W"], params["fc2"]["b"])))
    h = _linear(h, params["fc3"]["W"], params["fc3"]["b"])
    return (h, r)


# speedup vs baseline: 3.8408x; 1.0381x over previous
import functools

import jax
import jax.numpy as jnp
from jax import lax
from jax.experimental import pallas as pl
from jax.experimental.pallas import tpu as pltpu
from jax.experimental.pallas import tpu_sc as plsc

N_ROWS_LIN = 400
N_ROWS_DIST = 200
LANES = 128


def _ceil_to(x, m):
    return (x + m - 1) // m * m


def _linear_kernel(x_ref, wt_ref, b_ref, o_ref, *, act):
    y = jnp.dot(x_ref[...], wt_ref[...], preferred_element_type=jnp.float32)
    y = y + b_ref[...]
    if act == "relu":
        y = jnp.maximum(y, 0.0)
    o_ref[...] = y


def _linear(x, w, b, act=None, row_tile=N_ROWS_LIN):
    n, cin = x.shape
    cout = w.shape[0]
    assert n % row_tile == 0, (n, row_tile)
    wt = w.T
    b2 = b.reshape(1, cout)
    return pl.pallas_call(
        functools.partial(_linear_kernel, act=act),
        grid=(n // row_tile,),
        in_specs=[
            pl.BlockSpec((row_tile, cin), lambda i: (i, 0)),
            pl.BlockSpec((cin, cout), lambda i: (0, 0)),
            pl.BlockSpec((1, cout), lambda i: (0, 0)),
        ],
        out_specs=pl.BlockSpec((row_tile, cout), lambda i: (i, 0)),
        out_shape=jax.ShapeDtypeStruct((n, cout), jnp.float32),
    )(x, wt, b2)


def _dist_kernel(x_ref, xt_ref, o_ref, *, n_valid):
    x = x_ref[...]
    xt = xt_ref[...]
    dot = jnp.dot(x, xt, preferred_element_type=jnp.float32)
    sq_i = jnp.sum(x * x, axis=1, keepdims=True)
    sq_j = jnp.sum(xt * xt, axis=0, keepdims=True)
    neg = -((sq_i + sq_j) - 2.0 * dot)
    col = lax.broadcasted_iota(jnp.int32, neg.shape, 1)
    o_ref[...] = jnp.where(col < n_valid, neg, -1e30)


def _neg_dists(x):
    n, c = x.shape
    np_ = _ceil_to(n, LANES)
    xt = jnp.zeros((c, np_), jnp.float32).at[:, :n].set(x.T)
    return pl.pallas_call(
        functools.partial(_dist_kernel, n_valid=n),
        grid=(n // N_ROWS_DIST,),
        in_specs=[
            pl.BlockSpec((N_ROWS_DIST, c), lambda i: (i, 0)),
            pl.BlockSpec((c, np_), lambda i: (0, 0)),
        ],
        out_specs=pl.BlockSpec((N_ROWS_DIST, np_), lambda i: (i, 0)),
        out_shape=jax.ShapeDtypeStruct((n, np_), jnp.float32),
    )(x, xt)


def _knn(x, k):
    return _knn_fused(x, k)


_I32_MIN = -2147483648


def _sortable_keys(neg):
    b = lax.bitcast_convert_type(neg, jnp.int32)
    return jnp.where(b < 0, b ^ 0x7FFFFFFF, b)


def _knn_sel_kernel(x_ref, xt_ref, o_ref, *, n_valid, k, m, r_blk):
    x = x_ref[...]
    xt = xt_ref[...]
    npad = xt.shape[1]
    nc = npad // LANES
    dot = jnp.dot(x, xt, preferred_element_type=jnp.float32)
    sq_i = jnp.sum(x * x, axis=1, keepdims=True)
    sq_j = jnp.sum(xt * xt, axis=0, keepdims=True)
    neg = -((sq_i + sq_j) - 2.0 * dot)
    col = lax.broadcasted_iota(jnp.int32, neg.shape, 1)
    neg = jnp.where(col < n_valid, neg, -jnp.inf)
    key = _sortable_keys(neg)

    key3 = key.reshape(r_blk, nc, LANES)
    lane = lax.broadcasted_iota(jnp.int32, (r_blk, nc, LANES), 2)
    cbase = lax.broadcasted_iota(jnp.int32, (r_blk, nc), 1) * LANES
    cur = key3
    cvals, cidxs = [], []
    for _ in range(m):
        cmax = jnp.max(cur, axis=-1)
        amax = jnp.min(jnp.where(cur == cmax[..., None], lane, LANES), axis=-1)
        cvals.append(cmax)
        cidxs.append(cbase + amax)
        cur = jnp.where(lane == amax[..., None], _I32_MIN, cur)
    cand = jnp.concatenate(cvals, axis=1)
    cidx = jnp.concatenate(cidxs, axis=1)

    lo = jnp.full((r_blk, 1), _I32_MIN, jnp.int32)
    hi = jnp.full((r_blk, 1), 2147483647, jnp.int32)

    def bs_body(_, lh):
        lo, hi = lh
        d = (hi.astype(jnp.uint32) - lo.astype(jnp.uint32)) >> 1
        mid = (lo.astype(jnp.uint32) + d).astype(jnp.int32)
        cnt = jnp.sum((cand >= mid).astype(jnp.int32), axis=1, keepdims=True)
        ge = cnt >= k
        return jnp.where(ge, mid, lo), jnp.where(ge, hi, mid - 1)

    lo, hi = lax.fori_loop(0, 32, bs_body, (lo, hi))
    tau = lo

    cnt_full = jnp.sum((key >= tau).astype(jnp.int32), axis=1, keepdims=True)
    all_ok = jnp.all(cnt_full == k)

    p_iota = lax.broadcasted_iota(jnp.int32, (r_blk, k), 1)

    def fast_path():
        mask = cand >= tau
        w = cand.shape[1]
        tri = (lax.broadcasted_iota(jnp.int32, (w, w), 0)
               <= lax.broadcasted_iota(jnp.int32, (w, w), 1)).astype(jnp.float32)
        ranks = jnp.dot(mask.astype(jnp.float32), tri,
                        preferred_element_type=jnp.float32)
        ranks = jnp.where(mask, ranks, 0.0).astype(jnp.int32)
        cols = [jnp.sum(jnp.where(ranks == p + 1, cidx, 0), axis=1,
                        keepdims=True) for p in range(k)]
        return jnp.concatenate(cols, axis=1)

    def slow_path():
        colf = lax.broadcasted_iota(jnp.int32, key.shape, 1)

        def body(p, carry):
            kcur, out = carry
            vmax = jnp.max(kcur, axis=1, keepdims=True)
            j = jnp.min(jnp.where(kcur == vmax, colf, npad), axis=1,
                        keepdims=True)
            out = jnp.where(p_iota == p, j, out)
            kcur = jnp.where(colf == j, _I32_MIN, kcur)
            return kcur, out

        _, out = lax.fori_loop(0, k, body, (key, jnp.zeros((r_blk, k), jnp.int32)))
        return out

    o_ref[...] = lax.cond(all_ok, fast_path, slow_path)


def _knn_fused(x, k):
    n, c = x.shape
    np_ = _ceil_to(n, LANES)
    m = 6 if k <= 30 else 9
    r_blk = 200
    xt = jnp.zeros((c, np_), jnp.float32).at[:, :n].set(x.T)
    return pl.pallas_call(
        functools.partial(_knn_sel_kernel, n_valid=n, k=k, m=m, r_blk=r_blk),
        grid=(n // r_blk,),
        in_specs=[
            pl.BlockSpec((r_blk, c), lambda i: (i, 0)),
            pl.BlockSpec((c, np_), lambda i: (0, 0)),
        ],
        out_specs=pl.BlockSpec((r_blk, k), lambda i: (i, 0)),
        out_shape=jax.ShapeDtypeStruct((n, k), jnp.int32),
    )(x, xt)


_SC_WORKERS = 32


def _sc_gather(table, idx):
    v, d0 = table.shape
    d = _ceil_to(d0, LANES)
    if d != d0:
        table = jnp.zeros((v, d), table.dtype).at[:, :d0].set(table)
    b = idx.shape[0]
    chunk = max(64, min(1024, (100000 // d) // 8 * 8))
    blk = _SC_WORKERS * chunk
    b_pad = _ceil_to(b, blk)
    bpw = b_pad // _SC_WORKERS
    nch = bpw // chunk
    idx_pad = jnp.zeros((b_pad,), jnp.int32).at[:b].set(idx)

    mesh = plsc.VectorSubcoreMesh(core_axis_name="c", subcore_axis_name="s")

    @functools.partial(
        pl.kernel, mesh=mesh,
        out_type=jax.ShapeDtypeStruct((b_pad, d), jnp.float32),
        scratch_types=[
            pltpu.VMEM((chunk,), jnp.int32),
            pltpu.VMEM((chunk, d), jnp.float32),
            pltpu.SemaphoreType.DMA,
        ],
    )
    def gk(table_hbm, idx_hbm, out_hbm, idx_v, rows_v, sem):
        wid = lax.axis_index("s") * 2 + lax.axis_index("c")
        base = wid * bpw

        def body(ci, carry):
            off = base + ci * chunk
            pltpu.sync_copy(idx_hbm.at[pl.ds(off, chunk)], idx_v)
            pltpu.async_copy(table_hbm.at[idx_v], rows_v, sem).wait()
            pltpu.sync_copy(rows_v, out_hbm.at[pl.ds(off, chunk)])
            return carry

        lax.fori_loop(0, nch, body, 0)

    return gk(table, idx_pad)[:b, :d0]


def _neighbor_rows(x, idx):
    n, k = idx.shape
    if x.shape[1] >= 64:
        return _sc_gather(x, idx.reshape(-1)).reshape(n, k, x.shape[1])
    return x[idx]


def _edgeconv(p, idx, x):
    n, cin = x.shape
    k = idx.shape[1]
    cout = p["theta"]["W"].shape[0]
    diff = _neighbor_rows(x, idx) - x[:, None, :]
    msg = _linear(diff.reshape(n * k, cin), p["theta"]["W"], p["theta"]["b"])
    phi = _linear(x, p["phi"]["W"], p["phi"]["b"])
    m = msg.reshape(n, k, cout) + phi[:, None, :]
    return jnp.max(m, axis=1)


def _gatv2(p, idx, h, heads=4, out=256):
    n = h.shape[0]
    k = idx.shape[1]
    fs = _linear(h, p["src"]["W"], p["src"]["b"])
    fd = _linear(h, p["dst"]["W"], p["dst"]["b"]).reshape(n, heads, out)
    fsn = _sc_gather(fs, idx.reshape(-1)).reshape(n, k, heads, out)
    e = jax.nn.leaky_relu(fsn + fd[:, None], negative_slope=0.2)
    logits = jnp.sum(e * p["attn"][None, None], axis=-1)
    a = jax.nn.softmax(logits, axis=1)
    o = jnp.sum(a[..., None] * fsn, axis=1).reshape(n, heads * out)
    return o + p["bias"]


def _bn(p, x, eps=1e-5):
    m = jnp.mean(x, axis=0)
    v = jnp.var(x, axis=0)
    return p["gamma"] * (x - m) / jnp.sqrt(v + eps) + p["beta"]


def _tnet(p, x):
    h = _linear(x, p["fc1"]["W"], p["fc1"]["b"], act="relu")
    h = _linear(h, p["fc2"]["W"], p["fc2"]["b"])
    h = jnp.max(h, axis=0, keepdims=True)
    h = jax.nn.relu(h @ p["fc3"]["W"].T + p["fc3"]["b"])
    h = (h @ p["fc4"]["W"].T + p["fc4"]["b"]).reshape(3, 3)
    q, r = jnp.linalg.qr(h)
    return q, r


def kernel(pointcloud, params):
    t, r = _tnet(params["tnet"], pointcloud)
    x = pointcloud @ t
    idx3d = _knn(x, 30)
    h0 = _edgeconv(params["G1"], idx3d, x)
    h0 = _edgeconv(params["G2"], idx3d, h0)
    g0 = _knn(h0, 30)
    h1 = _edgeconv(params["G3"], g0, h0)
    h1 = _edgeconv(params["G4"], g0, h1)
    g1 = _knn(h1, 30)
    h2 = _edgeconv(params["G5"], g1, h1)
    h = jnp.concatenate([h0, h1, h2], axis=1)
    h_ = _linear(h, params["fc0"]["W"], params["fc0"]["b"])
    h_ = jnp.max(h_, axis=0, keepdims=True)
    h_ = jnp.tile(h_, (x.shape[0], 1))
    hxy = _edgeconv(params["E1xy"], _knn(x[:, 0:2], 30), x)
    hxy_ = _edgeconv(params["E2xy"], _knn(hxy, 100), hxy)
    hyz = _edgeconv(params["E1yz"], _knn(x[:, 1:3], 30), x)
    hyz_ = _edgeconv(params["E2yz"], _knn(hyz, 100), hyz)
    hxz = _edgeconv(params["E1xz"], _knn(x[:, ::2], 30), x)
    hxz_ = _edgeconv(params["E2xz"], _knn(hxz, 100), hxz)
    h = jnp.concatenate([h, h_, hxy_, hyz_, hxz_], axis=1)
    h = _gatv2(params["gat"], idx3d, h)
    h = jax.nn.relu(_bn(params["bn1"], _linear(h, params["fc1"]["W"], params["fc1"]["b"])))
    h = jax.nn.relu(_bn(params["bn2"], _linear(h, params["fc2"]["W"], params["fc2"]["b"])))
    h = _linear(h, params["fc3"]["W"], params["fc3"]["b"])
    return (h, r)
